# Initial kernel scaffold; baseline (speedup 1.0000x reference)
import jax
import jax.numpy as jnp
from jax.experimental import pallas as pl


def kernel(inputs, table, W1, b1, W2, b2):
    raise NotImplementedError("write your pallas kernel here")


# speedup vs baseline: 1.9956x; 1.9956x over previous
import functools

import jax
import jax.numpy as jnp
from jax import lax
from jax.experimental import pallas as pl
from jax.experimental.pallas import tpu as pltpu
from jax.experimental.pallas import tpu_sc as plsc

_NC = 2
_NS = 16
_NW = _NC * _NS

_B = 4096
_S = 200
_D = 32
_BPW = _B // _NW
_HALF = _S // 2

_mesh = plsc.VectorSubcoreMesh(
    core_axis_name="c", subcore_axis_name="s", num_cores=_NC, num_subcores=_NS
)


@functools.partial(
    pl.kernel,
    out_type=jax.ShapeDtypeStruct((_B, _D), jnp.float32),
    mesh=_mesh,
    compiler_params=pltpu.CompilerParams(use_tc_tiling_on_sc=False),
    scratch_types=[
        pltpu.VMEM((2 * _BPW, _HALF), jnp.int32),
        pltpu.VMEM((_HALF, _D), jnp.float32),
        pltpu.VMEM((_HALF, _D), jnp.float32),
        pltpu.VMEM((_BPW, _D), jnp.float32),
        pltpu.SemaphoreType.DMA,
    ],
)
def _pool_sum(idx_hbm, table_hbm, out_hbm, idx_v, buf_a, buf_b, acc_v, sem):
    cid = lax.axis_index("c")
    sid = lax.axis_index("s")
    wid = sid * _NC + cid
    pltpu.sync_copy(idx_hbm.at[wid], idx_v)

    def row_body(r, _):
        cp_a = pltpu.async_copy(table_hbm.at[idx_v.at[2 * r]], buf_a, sem)
        cp_b = pltpu.async_copy(table_hbm.at[idx_v.at[2 * r + 1]], buf_b, sem)
        cp_a.wait()
        cp_b.wait()

        def red(i, carry):
            s0, s1, s2, s3 = carry
            s0 = s0 + buf_a[i, 0:16]
            s1 = s1 + buf_a[i, 16:32]
            s2 = s2 + buf_b[i, 0:16]
            s3 = s3 + buf_b[i, 16:32]
            return (s0, s1, s2, s3)

        z = jnp.zeros((16,), jnp.float32)
        s0, s1, s2, s3 = lax.fori_loop(0, _HALF, red, (z, z, z, z))
        acc_v[r, 0:16] = s0 + s2
        acc_v[r, 16:32] = s1 + s3
        return 0

    lax.fori_loop(0, _BPW, row_body, 0)
    pltpu.sync_copy(acc_v, out_hbm.at[pl.ds(wid * _BPW, _BPW)])


_BB = 512


def _mlp_body(x_ref, w1_ref, b1_ref, w2_ref, b2_ref, o_ref):
    x = x_ref[...] * (1.0 / _S)
    h = jnp.dot(x, w1_ref[...], preferred_element_type=jnp.float32) + b1_ref[...]
    h = jnp.maximum(h, 0.0)
    logits = jnp.dot(h, w2_ref[...], preferred_element_type=jnp.float32) + b2_ref[...]
    m = jnp.max(logits, axis=-1, keepdims=True)
    e = jnp.exp(logits - m)
    o_ref[...] = e / jnp.sum(e, axis=-1, keepdims=True)


def _mlp(pooled_sum, W1, b1, W2, b2):
    n_classes = W2.shape[1]
    hidden = W1.shape[1]
    grid = (_B // _BB,)
    return pl.pallas_call(
        _mlp_body,
        grid=grid,
        in_specs=[
            pl.BlockSpec((_BB, _D), lambda i: (i, 0)),
            pl.BlockSpec((_D, hidden), lambda i: (0, 0)),
            pl.BlockSpec((1, hidden), lambda i: (0, 0)),
            pl.BlockSpec((hidden, n_classes), lambda i: (0, 0)),
            pl.BlockSpec((1, n_classes), lambda i: (0, 0)),
        ],
        out_specs=pl.BlockSpec((_BB, n_classes), lambda i: (i, 0)),
        out_shape=jax.ShapeDtypeStruct((_B, n_classes), jnp.float32),
    )(pooled_sum, W1, b1, W2, b2)


def kernel(inputs, table, W1, b1, W2, b2):
    idx = inputs.astype(jnp.int32).reshape(_NW, 2 * _BPW, _HALF)
    pooled_sum = _pool_sum(idx, table)
    return _mlp(pooled_sum, W1, b1.reshape(1, -1), W2, b2.reshape(1, -1))


# speedup vs baseline: 2.3785x; 1.1919x over previous
import functools

import jax
import jax.numpy as jnp
from jax import lax
from jax.experimental import pallas as pl
from jax.experimental.pallas import tpu as pltpu
from jax.experimental.pallas import tpu_sc as plsc

_NC = 2
_NS = 16
_NW = _NC * _NS

_B = 4096
_S = 200
_D = 32
_BPW = _B // _NW
_HALF = _S // 2

_mesh = plsc.VectorSubcoreMesh(
    core_axis_name="c", subcore_axis_name="s", num_cores=_NC, num_subcores=_NS
)


_NBUF = 4


@functools.partial(
    pl.kernel,
    out_type=jax.ShapeDtypeStruct((_B, _D), jnp.float32),
    mesh=_mesh,
    compiler_params=pltpu.CompilerParams(use_tc_tiling_on_sc=False),
    scratch_types=[
        pltpu.VMEM((2 * _BPW, _HALF), jnp.int32),
        [pltpu.VMEM((_S, _D), jnp.float32) for _ in range(_NBUF)],
        pltpu.VMEM((_BPW, _D), jnp.float32),
        [pltpu.SemaphoreType.DMA for _ in range(_NBUF)],
    ],
)
def _pool_sum(idx_hbm, table_hbm, out_hbm, idx_v, bufs, acc_v, sems):
    cid = lax.axis_index("c")
    sid = lax.axis_index("s")
    wid = sid * _NC + cid
    pltpu.sync_copy(idx_hbm.at[wid], idx_v)

    def issue(r, b):
        pltpu.async_copy(
            table_hbm.at[idx_v.at[2 * r]], bufs[b].at[pl.ds(0, _HALF)], sems[b]
        )
        pltpu.async_copy(
            table_hbm.at[idx_v.at[2 * r + 1]], bufs[b].at[pl.ds(_HALF, _HALF)], sems[b]
        )

    def drain(b):
        pltpu.make_async_copy(table_hbm.at[pl.ds(0, _S)], bufs[b], sems[b]).wait()

    def reduce_into(r, b):
        buf = bufs[b]

        def red(i, carry):
            s0, s1, s2, s3 = carry
            s0 = s0 + buf[i, 0:16]
            s1 = s1 + buf[i, 16:32]
            s2 = s2 + buf[_HALF + i, 0:16]
            s3 = s3 + buf[_HALF + i, 16:32]
            return (s0, s1, s2, s3)

        z = jnp.zeros((16,), jnp.float32)
        s0, s1, s2, s3 = lax.fori_loop(0, _HALF, red, (z, z, z, z))
        acc_v[r, 0:16] = s0 + s2
        acc_v[r, 16:32] = s1 + s3

    for b in range(_NBUF):
        issue(b, b)

    def body(r0, _):
        for b in range(_NBUF):
            r = r0 * _NBUF + b
            drain(b)
            reduce_into(r, b)
            issue(r + _NBUF, b)
        return 0

    lax.fori_loop(0, (_BPW - _NBUF) // _NBUF, body, 0)

    for b in range(_NBUF):
        r = _BPW - _NBUF + b
        drain(b)
        reduce_into(r, b)

    pltpu.sync_copy(acc_v, out_hbm.at[pl.ds(wid * _BPW, _BPW)])


_BB = 512


def _mlp_body(x_ref, w1_ref, b1_ref, w2_ref, b2_ref, o_ref):
    x = x_ref[...] * (1.0 / _S)
    h = jnp.dot(x, w1_ref[...], preferred_element_type=jnp.float32) + b1_ref[...]
    h = jnp.maximum(h, 0.0)
    logits = jnp.dot(h, w2_ref[...], preferred_element_type=jnp.float32) + b2_ref[...]
    m = jnp.max(logits, axis=-1, keepdims=True)
    e = jnp.exp(logits - m)
    o_ref[...] = e / jnp.sum(e, axis=-1, keepdims=True)


def _mlp(pooled_sum, W1, b1, W2, b2):
    n_classes = W2.shape[1]
    hidden = W1.shape[1]
    grid = (_B // _BB,)
    return pl.pallas_call(
        _mlp_body,
        grid=grid,
        in_specs=[
            pl.BlockSpec((_BB, _D), lambda i: (i, 0)),
            pl.BlockSpec((_D, hidden), lambda i: (0, 0)),
            pl.BlockSpec((1, hidden), lambda i: (0, 0)),
            pl.BlockSpec((hidden, n_classes), lambda i: (0, 0)),
            pl.BlockSpec((1, n_classes), lambda i: (0, 0)),
        ],
        out_specs=pl.BlockSpec((_BB, n_classes), lambda i: (i, 0)),
        out_shape=jax.ShapeDtypeStruct((_B, n_classes), jnp.float32),
    )(pooled_sum, W1, b1, W2, b2)


def kernel(inputs, table, W1, b1, W2, b2):
    idx = inputs.astype(jnp.int32).reshape(_NW, 2 * _BPW, _HALF)
    pooled_sum = _pool_sum(idx, table)
    return _mlp(pooled_sum, W1, b1.reshape(1, -1), W2, b2.reshape(1, -1))
